# Initial kernel scaffold; baseline (speedup 1.0000x reference)
import functools

import jax
import jax.numpy as jnp
from jax import lax
from jax.experimental import pallas as pl
from jax.experimental.pallas import tpu as pltpu
from jax.experimental.pallas import tpu_sc as plsc

N = 10000
E = 320000
D = 128
DH = 64
NUM_GRAPHS = 64
NUM_CLASS = 10

NTILES = 16
KE = 128
EPT = 20480
EP = NTILES * EPT
NCHUNK = EPT // KE
RPT = N // NTILES
NPAD = N + 16
ZROWS = NPAD // NTILES


def _sc_agg_body(m_hbm, ei_hbm, z_hbm, out_hbm, m_s, agg_s, src_v, dst_v,
                 rows_v, sem):
    c = lax.axis_index("c")
    s = lax.axis_index("s")
    col0 = c * DH
    r0 = s * RPT

    pltpu.sync_copy(m_hbm.at[pl.ds(r0, RPT), pl.ds(col0, DH)],
                    m_s.at[pl.ds(r0, RPT)])
    pltpu.sync_copy(z_hbm.at[pl.ds(0, ZROWS)], agg_s.at[pl.ds(s * ZROWS, ZROWS)])

    @pl.when(s == 0)
    def _():
        pltpu.sync_copy(z_hbm.at[pl.ds(0, 16)], m_s.at[pl.ds(N, 16)])

    plsc.subcore_barrier()

    ebase = s * EPT

    def chunk(i, carry):
        e0 = ebase + i * KE
        pltpu.sync_copy(ei_hbm.at[0, pl.ds(e0, KE)], src_v)
        pltpu.sync_copy(ei_hbm.at[1, pl.ds(e0, KE)], dst_v)
        pltpu.async_copy(m_s.at[src_v], rows_v, sem).wait()
        pltpu.sync_copy(rows_v, agg_s.at[dst_v], add=True)
        return carry

    lax.fori_loop(0, NCHUNK, chunk, 0)

    plsc.subcore_barrier()

    pltpu.sync_copy(agg_s.at[pl.ds(r0, RPT)],
                    out_hbm.at[pl.ds(r0, RPT), pl.ds(col0, DH)])


_sc_agg = functools.partial(
    pl.kernel,
    out_type=jax.ShapeDtypeStruct((N, D), jnp.float32),
    mesh=plsc.VectorSubcoreMesh(core_axis_name="c", subcore_axis_name="s"),
    scratch_types=[
        pltpu.VMEM_SHARED((NPAD, DH), jnp.float32),
        pltpu.VMEM_SHARED((NPAD, DH), jnp.float32),
        pltpu.VMEM((KE,), jnp.int32),
        pltpu.VMEM((KE,), jnp.int32),
        pltpu.VMEM((KE, DH), jnp.float32),
        pltpu.SemaphoreType.DMA,
    ],
)(_sc_agg_body)


_RB = 2000


def _mm0_body(x_ref, w_ref, m_ref):
    m_ref[...] = jnp.dot(x_ref[...], w_ref[...],
                         preferred_element_type=jnp.float32)


def _mm0(x, w):
    return pl.pallas_call(
        _mm0_body,
        grid=(N // _RB,),
        in_specs=[
            pl.BlockSpec((_RB, D), lambda r: (r, 0)),
            pl.BlockSpec((D, D), lambda r: (0, 0)),
        ],
        out_specs=pl.BlockSpec((_RB, D), lambda r: (r, 0)),
        out_shape=jax.ShapeDtypeStruct((N, D), jnp.float32),
    )(x, w)


def _gru_compute(agg, h, w_ih, w_hh, b_ih, b_hh):
    gi = lax.dot_general(agg, w_ih, (((1,), (1,)), ((), ())),
                         preferred_element_type=jnp.float32) + b_ih
    gh = lax.dot_general(h, w_hh, (((1,), (1,)), ((), ())),
                         preferred_element_type=jnp.float32) + b_hh
    r = jax.nn.sigmoid(gi[:, :D] + gh[:, :D])
    z = jax.nn.sigmoid(gi[:, D:2 * D] + gh[:, D:2 * D])
    n = jnp.tanh(gi[:, 2 * D:] + r * gh[:, 2 * D:])
    return (1.0 - z) * n + z * h


def _gru_mm_body(agg_ref, h_ref, wih_ref, whh_ref, bih_ref, bhh_ref, wn_ref,
                 h_out_ref, m_out_ref):
    hn = _gru_compute(agg_ref[...], h_ref[...], wih_ref[...], whh_ref[...],
                      bih_ref[...], bhh_ref[...])
    h_out_ref[...] = hn
    m_out_ref[...] = jnp.dot(hn, wn_ref[...], preferred_element_type=jnp.float32)


def _gru_mm(agg, h, w_ih, w_hh, b_ih, b_hh, w_next):
    return pl.pallas_call(
        _gru_mm_body,
        grid=(N // _RB,),
        in_specs=[
            pl.BlockSpec((_RB, D), lambda r: (r, 0)),
            pl.BlockSpec((_RB, D), lambda r: (r, 0)),
            pl.BlockSpec((3 * D, D), lambda r: (0, 0)),
            pl.BlockSpec((3 * D, D), lambda r: (0, 0)),
            pl.BlockSpec((1, 3 * D), lambda r: (0, 0)),
            pl.BlockSpec((1, 3 * D), lambda r: (0, 0)),
            pl.BlockSpec((D, D), lambda r: (0, 0)),
        ],
        out_specs=[
            pl.BlockSpec((_RB, D), lambda r: (r, 0)),
            pl.BlockSpec((_RB, D), lambda r: (r, 0)),
        ],
        out_shape=[
            jax.ShapeDtypeStruct((N, D), jnp.float32),
            jax.ShapeDtypeStruct((N, D), jnp.float32),
        ],
    )(agg, h, w_ih, w_hh, b_ih, b_hh, w_next)


def _gru_final_body(agg_ref, h_ref, wih_ref, whh_ref, bih_ref, bhh_ref,
                    out_ref):
    hn = _gru_compute(agg_ref[...], h_ref[...], wih_ref[...], whh_ref[...],
                      bih_ref[...], bhh_ref[...])
    norm = jnp.maximum(jnp.sqrt(jnp.sum(hn * hn, axis=1, keepdims=True)),
                       1e-12)
    out_ref[...] = jnp.maximum(hn / norm, 0.0)


def _gru_final(agg, h, w_ih, w_hh, b_ih, b_hh):
    return pl.pallas_call(
        _gru_final_body,
        grid=(N // _RB,),
        in_specs=[
            pl.BlockSpec((_RB, D), lambda r: (r, 0)),
            pl.BlockSpec((_RB, D), lambda r: (r, 0)),
            pl.BlockSpec((3 * D, D), lambda r: (0, 0)),
            pl.BlockSpec((3 * D, D), lambda r: (0, 0)),
            pl.BlockSpec((1, 3 * D), lambda r: (0, 0)),
            pl.BlockSpec((1, 3 * D), lambda r: (0, 0)),
        ],
        out_specs=pl.BlockSpec((_RB, D), lambda r: (r, 0)),
        out_shape=jax.ShapeDtypeStruct((N, D), jnp.float32),
    )(agg, h, w_ih, w_hh, b_ih, b_hh)


def _pool_body(x_ref, b_ref, lw_ref, lb_ref, out_ref):
    g = pl.program_id(0)
    x = x_ref[...]
    mask = b_ref[...] == g
    mx = jnp.max(jnp.where(mask, x, -jnp.inf), axis=0, keepdims=True)
    sm = jnp.sum(jnp.where(mask, x, 0.0), axis=0, keepdims=True)
    cnt = jnp.sum(mask.astype(jnp.float32))
    mean = sm / jnp.maximum(cnt, 1.0)
    pooled = jnp.concatenate([mx, mean], axis=1)
    out_ref[...] = lax.dot_general(pooled, lw_ref[...],
                                   (((1,), (1,)), ((), ())),
                                   preferred_element_type=jnp.float32) \
        + lb_ref[...]


def _pool(x, batch2d, lin_w, lin_b):
    return pl.pallas_call(
        _pool_body,
        grid=(NUM_GRAPHS,),
        in_specs=[
            pl.BlockSpec((N, D), lambda g: (0, 0)),
            pl.BlockSpec((N, 1), lambda g: (0, 0)),
            pl.BlockSpec((NUM_CLASS, 2 * D), lambda g: (0, 0)),
            pl.BlockSpec((1, NUM_CLASS), lambda g: (0, 0)),
        ],
        out_specs=pl.BlockSpec((1, NUM_CLASS), lambda g: (g, 0)),
        out_shape=jax.ShapeDtypeStruct((NUM_GRAPHS, NUM_CLASS), jnp.float32),
    )(x, batch2d, lin_w, lin_b)


def kernel(x, edge_index, batch, weight, w_ih, w_hh, b_ih, b_hh, lin_w, lin_b):
    ei = edge_index.astype(jnp.int32)
    pad = EP - E
    src = jnp.concatenate([ei[0], jnp.full((pad,), N, jnp.int32)])
    dst = jnp.concatenate([ei[1], jnp.full((pad,), N + 8, jnp.int32)])
    ei_p = jnp.stack([src, dst])
    zeros = jnp.zeros((ZROWS, DH), jnp.float32)
    batch2d = batch.astype(jnp.int32).reshape(N, 1)
    b_ih2 = b_ih.reshape(1, 3 * D)
    b_hh2 = b_hh.reshape(1, 3 * D)
    lin_b2 = lin_b.reshape(1, NUM_CLASS)

    h = x
    m = _mm0(x, weight[0])
    for i in range(3):
        agg = _sc_agg(m, ei_p, zeros)
        if i < 2:
            h, m = _gru_mm(agg, h, w_ih, w_hh, b_ih2, b_hh2, weight[i + 1])
        else:
            out = _gru_final(agg, h, w_ih, w_hh, b_ih2, b_hh2)
    return _pool(out, batch2d, lin_w, lin_b2)


# speedup vs baseline: 2.2839x; 2.2839x over previous
import functools

import jax
import jax.numpy as jnp
from jax import lax
from jax.experimental import pallas as pl
from jax.experimental.pallas import tpu as pltpu
from jax.experimental.pallas import tpu_sc as plsc

N = 10000
E = 320000
D = 128
NUM_GRAPHS = 64
NUM_CLASS = 10

NTILES = 16
KE = 128
EP = 327680
EPC = EP // 2
EPT = EPC // NTILES
NCHUNK = EPT // KE
RPT = 624
NPAD = N + 16
ZROWS = RPT


def _sc_agg_body(m_hbm, ei_hbm, z_hbm, out_hbm, agg_s, src_v, dst_v,
                 rows_v, sem):
    c = lax.axis_index("c")
    s = lax.axis_index("s")
    r0 = s * RPT

    pltpu.sync_copy(z_hbm.at[pl.ds(0, RPT)], agg_s.at[pl.ds(r0, RPT)])

    @pl.when(s == 0)
    def _():
        pltpu.sync_copy(z_hbm.at[pl.ds(0, 32)],
                        agg_s.at[pl.ds(16 * RPT, 32)])

    plsc.subcore_barrier()

    ebase = c * EPC + s * EPT

    def chunk(i, carry):
        e0 = ebase + i * KE
        pltpu.sync_copy(ei_hbm.at[0, pl.ds(e0, KE)], src_v)
        pltpu.sync_copy(ei_hbm.at[1, pl.ds(e0, KE)], dst_v)
        pltpu.async_copy(m_hbm.at[src_v], rows_v, sem).wait()
        pltpu.sync_copy(rows_v, agg_s.at[dst_v], add=True)
        return carry

    lax.fori_loop(0, NCHUNK, chunk, 0)

    plsc.subcore_barrier()

    pltpu.sync_copy(agg_s.at[pl.ds(r0, RPT)], out_hbm.at[c, pl.ds(r0, RPT)])

    @pl.when(s == 15)
    def _():
        pltpu.sync_copy(agg_s.at[pl.ds(16 * RPT, 16)],
                        out_hbm.at[c, pl.ds(16 * RPT, 16)])


@functools.cache
def _sc_agg_kernel():
    return pl.kernel(
        _sc_agg_body,
        out_type=jax.ShapeDtypeStruct((2, N, D), jnp.float32),
        mesh=plsc.VectorSubcoreMesh(core_axis_name="c", subcore_axis_name="s"),
        scratch_types=[
            pltpu.VMEM_SHARED((NPAD, D), jnp.float32),
            pltpu.VMEM((KE,), jnp.int32),
            pltpu.VMEM((KE,), jnp.int32),
            pltpu.VMEM((KE, D), jnp.float32),
            pltpu.SemaphoreType.DMA,
        ],
    )


def _sc_agg(m, ei_p, zeros):
    return _sc_agg_kernel()(m, ei_p, zeros)


_RB = 2000


def _mm0_body(x_ref, w_ref, m_ref):
    m_ref[...] = jnp.dot(x_ref[...], w_ref[...],
                         preferred_element_type=jnp.float32)


def _mm0(x, w):
    return pl.pallas_call(
        _mm0_body,
        grid=(N // _RB,),
        in_specs=[
            pl.BlockSpec((_RB, D), lambda r: (r, 0)),
            pl.BlockSpec((D, D), lambda r: (0, 0)),
        ],
        out_specs=pl.BlockSpec((_RB, D), lambda r: (r, 0)),
        out_shape=jax.ShapeDtypeStruct((N, D), jnp.float32),
    )(x, w)


def _gru_compute(aggp_ref, h, w_ih, w_hh, b_ih, b_hh):
    agg = aggp_ref[0, :, :] + aggp_ref[1, :, :]
    gi = lax.dot_general(agg, w_ih, (((1,), (1,)), ((), ())),
                         preferred_element_type=jnp.float32) + b_ih
    gh = lax.dot_general(h, w_hh, (((1,), (1,)), ((), ())),
                         preferred_element_type=jnp.float32) + b_hh
    r = jax.nn.sigmoid(gi[:, :D] + gh[:, :D])
    z = jax.nn.sigmoid(gi[:, D:2 * D] + gh[:, D:2 * D])
    n = jnp.tanh(gi[:, 2 * D:] + r * gh[:, 2 * D:])
    return (1.0 - z) * n + z * h


def _gru_mm_body(agg_ref, h_ref, wih_ref, whh_ref, bih_ref, bhh_ref, wn_ref,
                 h_out_ref, m_out_ref):
    hn = _gru_compute(agg_ref, h_ref[...], wih_ref[...], whh_ref[...],
                      bih_ref[...], bhh_ref[...])
    h_out_ref[...] = hn
    m_out_ref[...] = jnp.dot(hn, wn_ref[...],
                             preferred_element_type=jnp.float32)


def _gru_mm(aggp, h, w_ih, w_hh, b_ih, b_hh, w_next):
    return pl.pallas_call(
        _gru_mm_body,
        grid=(N // _RB,),
        in_specs=[
            pl.BlockSpec((2, _RB, D), lambda r: (0, r, 0)),
            pl.BlockSpec((_RB, D), lambda r: (r, 0)),
            pl.BlockSpec((3 * D, D), lambda r: (0, 0)),
            pl.BlockSpec((3 * D, D), lambda r: (0, 0)),
            pl.BlockSpec((1, 3 * D), lambda r: (0, 0)),
            pl.BlockSpec((1, 3 * D), lambda r: (0, 0)),
            pl.BlockSpec((D, D), lambda r: (0, 0)),
        ],
        out_specs=[
            pl.BlockSpec((_RB, D), lambda r: (r, 0)),
            pl.BlockSpec((_RB, D), lambda r: (r, 0)),
        ],
        out_shape=[
            jax.ShapeDtypeStruct((N, D), jnp.float32),
            jax.ShapeDtypeStruct((N, D), jnp.float32),
        ],
    )(aggp, h, w_ih, w_hh, b_ih, b_hh, w_next)


def _gru_final_body(agg_ref, h_ref, wih_ref, whh_ref, bih_ref, bhh_ref,
                    out_ref):
    hn = _gru_compute(agg_ref, h_ref[...], wih_ref[...], whh_ref[...],
                      bih_ref[...], bhh_ref[...])
    norm = jnp.maximum(jnp.sqrt(jnp.sum(hn * hn, axis=1, keepdims=True)),
                       1e-12)
    out_ref[...] = jnp.maximum(hn / norm, 0.0)


def _gru_final(aggp, h, w_ih, w_hh, b_ih, b_hh):
    return pl.pallas_call(
        _gru_final_body,
        grid=(N // _RB,),
        in_specs=[
            pl.BlockSpec((2, _RB, D), lambda r: (0, r, 0)),
            pl.BlockSpec((_RB, D), lambda r: (r, 0)),
            pl.BlockSpec((3 * D, D), lambda r: (0, 0)),
            pl.BlockSpec((3 * D, D), lambda r: (0, 0)),
            pl.BlockSpec((1, 3 * D), lambda r: (0, 0)),
            pl.BlockSpec((1, 3 * D), lambda r: (0, 0)),
        ],
        out_specs=pl.BlockSpec((_RB, D), lambda r: (r, 0)),
        out_shape=jax.ShapeDtypeStruct((N, D), jnp.float32),
    )(aggp, h, w_ih, w_hh, b_ih, b_hh)


_GPB = 8


def _pool_body(x_ref, b_ref, lw_ref, lb_ref, out_ref):
    p = pl.program_id(0)
    x = x_ref[...]
    b = b_ref[...]
    rows = []
    for j in range(_GPB):
        mask = b == (p * _GPB + j)
        mx = jnp.max(jnp.where(mask, x, -jnp.inf), axis=0, keepdims=True)
        sm = jnp.sum(jnp.where(mask, x, 0.0), axis=0, keepdims=True)
        cnt = jnp.sum(mask.astype(jnp.float32))
        rows.append(jnp.concatenate([mx, sm / jnp.maximum(cnt, 1.0)], axis=1))
    pooled = jnp.concatenate(rows, axis=0)
    out_ref[...] = lax.dot_general(pooled, lw_ref[...],
                                   (((1,), (1,)), ((), ())),
                                   preferred_element_type=jnp.float32) \
        + lb_ref[...]


def _pool(x, batch2d, lin_w, lin_b):
    return pl.pallas_call(
        _pool_body,
        grid=(NUM_GRAPHS // _GPB,),
        in_specs=[
            pl.BlockSpec((N, D), lambda g: (0, 0)),
            pl.BlockSpec((N, 1), lambda g: (0, 0)),
            pl.BlockSpec((NUM_CLASS, 2 * D), lambda g: (0, 0)),
            pl.BlockSpec((1, NUM_CLASS), lambda g: (0, 0)),
        ],
        out_specs=pl.BlockSpec((_GPB, NUM_CLASS), lambda g: (g, 0)),
        out_shape=jax.ShapeDtypeStruct((NUM_GRAPHS, NUM_CLASS), jnp.float32),
    )(x, batch2d, lin_w, lin_b)


def kernel(x, edge_index, batch, weight, w_ih, w_hh, b_ih, b_hh, lin_w, lin_b):
    ei = edge_index.astype(jnp.int32)
    pad = EP - E
    src = jnp.concatenate([ei[0], jnp.zeros((pad,), jnp.int32)])
    dst = jnp.concatenate(
        [ei[1], N + (jnp.arange(pad, dtype=jnp.int32) % 16)])
    ei_p = jnp.stack([src, dst])
    zeros = jnp.zeros((ZROWS, D), jnp.float32)
    batch2d = batch.astype(jnp.int32).reshape(N, 1)
    b_ih2 = b_ih.reshape(1, 3 * D)
    b_hh2 = b_hh.reshape(1, 3 * D)
    lin_b2 = lin_b.reshape(1, NUM_CLASS)

    h = x
    m = _mm0(x, weight[0])
    for i in range(3):
        aggp = _sc_agg(m, ei_p, zeros)
        if i < 2:
            h, m = _gru_mm(aggp, h, w_ih, w_hh, b_ih2, b_hh2, weight[i + 1])
        else:
            out = _gru_final(aggp, h, w_ih, w_hh, b_ih2, b_hh2)
    return _pool(out, batch2d, lin_w, lin_b2)


# speedup vs baseline: 2.6889x; 1.1773x over previous
import functools

import jax
import jax.numpy as jnp
from jax import lax
from jax.experimental import pallas as pl
from jax.experimental.pallas import tpu as pltpu
from jax.experimental.pallas import tpu_sc as plsc

N = 10000
E = 320000
D = 128
NUM_GRAPHS = 64
NUM_CLASS = 10

NTILES = 16
KE = 128
EP = 327680
EPC = EP // 2
EPT = EPC // NTILES
NCHUNK = EPT // KE
RPT = 624
NPAD = N + 16
ZROWS = RPT


def _sc_agg_body(m_hbm, ei_hbm, z_hbm, out_hbm, agg_s, idxr, rows,
                 sem_i, sem_g, sem_s):
    c = lax.axis_index("c")
    s = lax.axis_index("s")
    r0 = s * RPT

    pltpu.sync_copy(z_hbm.at[pl.ds(0, RPT)], agg_s.at[pl.ds(r0, RPT)])

    @pl.when(s == 0)
    def _():
        pltpu.sync_copy(z_hbm.at[pl.ds(0, 32)],
                        agg_s.at[pl.ds(16 * RPT, 32)])

    plsc.subcore_barrier()

    c0 = (c * NTILES + s) * NCHUNK

    def idx_start(i, q):
        pltpu.async_copy(ei_hbm.at[c0 + i], idxr.at[q], sem_i.at[q])

    def idx_wait(q):
        pltpu.make_async_copy(ei_hbm.at[c0], idxr.at[q], sem_i.at[q]).wait()

    def g_start(q, b):
        pltpu.async_copy(m_hbm.at[idxr.at[q, 0]], rows.at[b], sem_g.at[b])

    def g_wait(q, b):
        pltpu.make_async_copy(m_hbm.at[idxr.at[q, 0]], rows.at[b],
                              sem_g.at[b]).wait()

    def s_start(q, b):
        pltpu.async_copy(rows.at[b], agg_s.at[idxr.at[q, 1]], sem_s.at[b],
                         add=True)

    def s_wait(q, b):
        pltpu.make_async_copy(rows.at[b], agg_s.at[idxr.at[q, 1]],
                              sem_s.at[b]).wait()

    def chunk_steps(i, k, first, last):
        q, q2, b = k % 4, (k + 2) % 4, k % 2
        if not first:
            s_wait(q2, b)
        if not last:
            idx_start(i + 2, q2)
        idx_wait(q)
        g_start(q, b)
        g_wait(q, b)
        s_start(q, b)

    idx_start(0, 0)
    idx_start(1, 1)
    for k in range(4):
        chunk_steps(k, k, first=(k < 2), last=False)

    def group(j, carry):
        i0 = j * 4
        for k in range(4):
            chunk_steps(i0 + k, k, first=False, last=False)
        return carry

    lax.fori_loop(1, NCHUNK // 4 - 1, group, 0)

    for k in range(4):
        i = NCHUNK - 4 + k
        chunk_steps(i, k, first=False, last=(i + 2 >= NCHUNK))
    s_wait(2, 0)
    s_wait(3, 1)

    plsc.subcore_barrier()

    pltpu.sync_copy(agg_s.at[pl.ds(r0, RPT)], out_hbm.at[c, pl.ds(r0, RPT)])

    @pl.when(s == 15)
    def _():
        pltpu.sync_copy(agg_s.at[pl.ds(16 * RPT, 16)],
                        out_hbm.at[c, pl.ds(16 * RPT, 16)])


@functools.cache
def _sc_agg_kernel():
    return pl.kernel(
        _sc_agg_body,
        out_type=jax.ShapeDtypeStruct((2, N, D), jnp.float32),
        mesh=plsc.VectorSubcoreMesh(core_axis_name="c", subcore_axis_name="s"),
        scratch_types=[
            pltpu.VMEM_SHARED((NPAD, D), jnp.float32),
            pltpu.VMEM((4, 2, KE), jnp.int32),
            pltpu.VMEM((2, KE, D), jnp.float32),
            pltpu.SemaphoreType.DMA((4,)),
            pltpu.SemaphoreType.DMA((2,)),
            pltpu.SemaphoreType.DMA((2,)),
        ],
    )


def _sc_agg(m, ei_p, zeros):
    return _sc_agg_kernel()(m, ei_p, zeros)


_RB = 2000


def _mm0_body(x_ref, w_ref, m_ref):
    m_ref[...] = jnp.dot(x_ref[...], w_ref[...],
                         preferred_element_type=jnp.float32)


def _mm0(x, w):
    return pl.pallas_call(
        _mm0_body,
        grid=(N // _RB,),
        in_specs=[
            pl.BlockSpec((_RB, D), lambda r: (r, 0)),
            pl.BlockSpec((D, D), lambda r: (0, 0)),
        ],
        out_specs=pl.BlockSpec((_RB, D), lambda r: (r, 0)),
        out_shape=jax.ShapeDtypeStruct((N, D), jnp.float32),
    )(x, w)


def _gru_compute(aggp_ref, h, w_ih, w_hh, b_ih, b_hh):
    agg = aggp_ref[0, :, :] + aggp_ref[1, :, :]
    gi = lax.dot_general(agg, w_ih, (((1,), (1,)), ((), ())),
                         preferred_element_type=jnp.float32) + b_ih
    gh = lax.dot_general(h, w_hh, (((1,), (1,)), ((), ())),
                         preferred_element_type=jnp.float32) + b_hh
    r = jax.nn.sigmoid(gi[:, :D] + gh[:, :D])
    z = jax.nn.sigmoid(gi[:, D:2 * D] + gh[:, D:2 * D])
    n = jnp.tanh(gi[:, 2 * D:] + r * gh[:, 2 * D:])
    return (1.0 - z) * n + z * h


def _gru_mm_body(agg_ref, h_ref, wih_ref, whh_ref, bih_ref, bhh_ref, wn_ref,
                 h_out_ref, m_out_ref):
    hn = _gru_compute(agg_ref, h_ref[...], wih_ref[...], whh_ref[...],
                      bih_ref[...], bhh_ref[...])
    h_out_ref[...] = hn
    m_out_ref[...] = jnp.dot(hn, wn_ref[...],
                             preferred_element_type=jnp.float32)


def _gru_mm(aggp, h, w_ih, w_hh, b_ih, b_hh, w_next):
    return pl.pallas_call(
        _gru_mm_body,
        grid=(N // _RB,),
        in_specs=[
            pl.BlockSpec((2, _RB, D), lambda r: (0, r, 0)),
            pl.BlockSpec((_RB, D), lambda r: (r, 0)),
            pl.BlockSpec((3 * D, D), lambda r: (0, 0)),
            pl.BlockSpec((3 * D, D), lambda r: (0, 0)),
            pl.BlockSpec((1, 3 * D), lambda r: (0, 0)),
            pl.BlockSpec((1, 3 * D), lambda r: (0, 0)),
            pl.BlockSpec((D, D), lambda r: (0, 0)),
        ],
        out_specs=[
            pl.BlockSpec((_RB, D), lambda r: (r, 0)),
            pl.BlockSpec((_RB, D), lambda r: (r, 0)),
        ],
        out_shape=[
            jax.ShapeDtypeStruct((N, D), jnp.float32),
            jax.ShapeDtypeStruct((N, D), jnp.float32),
        ],
    )(aggp, h, w_ih, w_hh, b_ih, b_hh, w_next)


def _gru_final_body(agg_ref, h_ref, wih_ref, whh_ref, bih_ref, bhh_ref,
                    out_ref):
    hn = _gru_compute(agg_ref, h_ref[...], wih_ref[...], whh_ref[...],
                      bih_ref[...], bhh_ref[...])
    norm = jnp.maximum(jnp.sqrt(jnp.sum(hn * hn, axis=1, keepdims=True)),
                       1e-12)
    out_ref[...] = jnp.maximum(hn / norm, 0.0)


def _gru_final(aggp, h, w_ih, w_hh, b_ih, b_hh):
    return pl.pallas_call(
        _gru_final_body,
        grid=(N // _RB,),
        in_specs=[
            pl.BlockSpec((2, _RB, D), lambda r: (0, r, 0)),
            pl.BlockSpec((_RB, D), lambda r: (r, 0)),
            pl.BlockSpec((3 * D, D), lambda r: (0, 0)),
            pl.BlockSpec((3 * D, D), lambda r: (0, 0)),
            pl.BlockSpec((1, 3 * D), lambda r: (0, 0)),
            pl.BlockSpec((1, 3 * D), lambda r: (0, 0)),
        ],
        out_specs=pl.BlockSpec((_RB, D), lambda r: (r, 0)),
        out_shape=jax.ShapeDtypeStruct((N, D), jnp.float32),
    )(aggp, h, w_ih, w_hh, b_ih, b_hh)


_GPB = 8


def _pool_body(x_ref, b_ref, lw_ref, lb_ref, out_ref):
    p = pl.program_id(0)
    x = x_ref[...]
    b = b_ref[...]
    rows = []
    for j in range(_GPB):
        mask = b == (p * _GPB + j)
        mx = jnp.max(jnp.where(mask, x, -jnp.inf), axis=0, keepdims=True)
        sm = jnp.sum(jnp.where(mask, x, 0.0), axis=0, keepdims=True)
        cnt = jnp.sum(mask.astype(jnp.float32))
        rows.append(jnp.concatenate([mx, sm / jnp.maximum(cnt, 1.0)], axis=1))
    pooled = jnp.concatenate(rows, axis=0)
    out_ref[...] = lax.dot_general(pooled, lw_ref[...],
                                   (((1,), (1,)), ((), ())),
                                   preferred_element_type=jnp.float32) \
        + lb_ref[...]


def _pool(x, batch2d, lin_w, lin_b):
    return pl.pallas_call(
        _pool_body,
        grid=(NUM_GRAPHS // _GPB,),
        in_specs=[
            pl.BlockSpec((N, D), lambda g: (0, 0)),
            pl.BlockSpec((N, 1), lambda g: (0, 0)),
            pl.BlockSpec((NUM_CLASS, 2 * D), lambda g: (0, 0)),
            pl.BlockSpec((1, NUM_CLASS), lambda g: (0, 0)),
        ],
        out_specs=pl.BlockSpec((_GPB, NUM_CLASS), lambda g: (g, 0)),
        out_shape=jax.ShapeDtypeStruct((NUM_GRAPHS, NUM_CLASS), jnp.float32),
    )(x, batch2d, lin_w, lin_b)


def kernel(x, edge_index, batch, weight, w_ih, w_hh, b_ih, b_hh, lin_w, lin_b):
    ei = edge_index.astype(jnp.int32)
    pad = EP - E
    src = jnp.concatenate([ei[0], jnp.zeros((pad,), jnp.int32)])
    dst = jnp.concatenate(
        [ei[1], N + (jnp.arange(pad, dtype=jnp.int32) % 16)])
    ei_p = jnp.stack([src.reshape(EP // KE, KE),
                      dst.reshape(EP // KE, KE)], axis=1)
    zeros = jnp.zeros((ZROWS, D), jnp.float32)
    batch2d = batch.astype(jnp.int32).reshape(N, 1)
    b_ih2 = b_ih.reshape(1, 3 * D)
    b_hh2 = b_hh.reshape(1, 3 * D)
    lin_b2 = lin_b.reshape(1, NUM_CLASS)

    h = x
    m = _mm0(x, weight[0])
    for i in range(3):
        aggp = _sc_agg(m, ei_p, zeros)
        if i < 2:
            h, m = _gru_mm(aggp, h, w_ih, w_hh, b_ih2, b_hh2, weight[i + 1])
        else:
            out = _gru_final(aggp, h, w_ih, w_hh, b_ih2, b_hh2)
    return _pool(out, batch2d, lin_w, lin_b2)


# speedup vs baseline: 2.7581x; 1.0257x over previous
import functools

import jax
import jax.numpy as jnp
from jax import lax
from jax.experimental import pallas as pl
from jax.experimental.pallas import tpu as pltpu
from jax.experimental.pallas import tpu_sc as plsc

N = 10000
E = 320000
D = 128
NUM_GRAPHS = 64
NUM_CLASS = 10

NTILES = 16
KE = 128
EP = 327680
EPC = EP // 2
EPT = EPC // NTILES
NCHUNK = EPT // KE
RPT = 624
NPAD = N + 16
ZROWS = RPT


def _sc_agg_body(m_hbm, ei_hbm, z_hbm, out_hbm, agg_s, idxr, rows,
                 sem_i, sem_g, sem_s):
    c = lax.axis_index("c")
    s = lax.axis_index("s")
    r0 = s * RPT

    pltpu.sync_copy(z_hbm.at[pl.ds(0, RPT)], agg_s.at[pl.ds(r0, RPT)])

    @pl.when(s == 0)
    def _():
        pltpu.sync_copy(z_hbm.at[pl.ds(0, 32)],
                        agg_s.at[pl.ds(16 * RPT, 32)])

    plsc.subcore_barrier()

    c0 = (c * NTILES + s) * NCHUNK

    def idx_start(i, q):
        pltpu.async_copy(ei_hbm.at[c0 + i], idxr.at[q], sem_i.at[q])

    def idx_wait(q):
        pltpu.make_async_copy(ei_hbm.at[c0], idxr.at[q], sem_i.at[q]).wait()

    def g_start(q, b):
        pltpu.async_copy(m_hbm.at[idxr.at[q, 0]], rows.at[b], sem_g.at[b])

    def g_wait(q, b):
        pltpu.make_async_copy(m_hbm.at[idxr.at[q, 0]], rows.at[b],
                              sem_g.at[b]).wait()

    def s_start(q, b):
        pltpu.async_copy(rows.at[b], agg_s.at[idxr.at[q, 1]], sem_s.at[b],
                         add=True)

    def s_wait(q, b):
        pltpu.make_async_copy(rows.at[b], agg_s.at[idxr.at[q, 1]],
                              sem_s.at[b]).wait()

    def chunk_steps(i, im, first, no_pref, no_g):
        q, q1, q2 = im % 4, (im + 1) % 4, (im + 2) % 4
        b, b1 = im % 3, (im + 1) % 3
        if not first:
            s_wait(q2, b1)
        if not no_pref:
            idx_start(i + 2, q2)
        if not no_g:
            idx_wait(q1)
            g_start(q1, b1)
        g_wait(q, b)
        s_start(q, b)

    idx_start(0, 0)
    idx_start(1, 1)
    idx_wait(0)
    g_start(0, 0)
    for k in range(4):
        chunk_steps(k, k, first=(k < 2), no_pref=False, no_g=False)

    def group(jj, carry):
        i0 = 4 + jj * 12
        for k in range(12):
            chunk_steps(i0 + k, 4 + k, first=False, no_pref=False,
                        no_g=False)
        return carry

    lax.fori_loop(0, (NCHUNK - 8) // 12, group, 0)

    for k in range(4):
        i = NCHUNK - 4 + k
        chunk_steps(i, i, first=False, no_pref=(i + 2 >= NCHUNK),
                    no_g=(i + 1 >= NCHUNK))
    for i in (NCHUNK - 2, NCHUNK - 1):
        s_wait(i % 4, i % 3)

    plsc.subcore_barrier()

    pltpu.sync_copy(agg_s.at[pl.ds(r0, RPT)], out_hbm.at[c, pl.ds(r0, RPT)])

    @pl.when(s == 15)
    def _():
        pltpu.sync_copy(agg_s.at[pl.ds(16 * RPT, 16)],
                        out_hbm.at[c, pl.ds(16 * RPT, 16)])


@functools.cache
def _sc_agg_kernel():
    return pl.kernel(
        _sc_agg_body,
        out_type=jax.ShapeDtypeStruct((2, N, D), jnp.float32),
        mesh=plsc.VectorSubcoreMesh(core_axis_name="c", subcore_axis_name="s"),
        scratch_types=[
            pltpu.VMEM_SHARED((NPAD, D), jnp.float32),
            pltpu.VMEM((4, 2, KE), jnp.int32),
            pltpu.VMEM((3, KE, D), jnp.float32),
            pltpu.SemaphoreType.DMA((4,)),
            pltpu.SemaphoreType.DMA((3,)),
            pltpu.SemaphoreType.DMA((3,)),
        ],
    )


def _sc_agg(m, ei_p, zeros):
    return _sc_agg_kernel()(m, ei_p, zeros)


_RB = 2000


def _mm0_body(x_ref, w_ref, m_ref):
    m_ref[...] = jnp.dot(x_ref[...], w_ref[...],
                         preferred_element_type=jnp.float32)


def _mm0(x, w):
    return pl.pallas_call(
        _mm0_body,
        grid=(N // _RB,),
        in_specs=[
            pl.BlockSpec((_RB, D), lambda r: (r, 0)),
            pl.BlockSpec((D, D), lambda r: (0, 0)),
        ],
        out_specs=pl.BlockSpec((_RB, D), lambda r: (r, 0)),
        out_shape=jax.ShapeDtypeStruct((N, D), jnp.float32),
    )(x, w)


def _gru_compute(aggp_ref, h, w_ih, w_hh, b_ih, b_hh):
    agg = aggp_ref[0, :, :].astype(jnp.float32) \
        + aggp_ref[1, :, :].astype(jnp.float32)
    gi = lax.dot_general(agg, w_ih, (((1,), (1,)), ((), ())),
                         preferred_element_type=jnp.float32) + b_ih
    gh = lax.dot_general(h, w_hh, (((1,), (1,)), ((), ())),
                         preferred_element_type=jnp.float32) + b_hh
    r = jax.nn.sigmoid(gi[:, :D] + gh[:, :D])
    z = jax.nn.sigmoid(gi[:, D:2 * D] + gh[:, D:2 * D])
    n = jnp.tanh(gi[:, 2 * D:] + r * gh[:, 2 * D:])
    return (1.0 - z) * n + z * h


def _gru_mm_body(agg_ref, h_ref, wih_ref, whh_ref, bih_ref, bhh_ref, wn_ref,
                 h_out_ref, m_out_ref):
    hn = _gru_compute(agg_ref, h_ref[...], wih_ref[...], whh_ref[...],
                      bih_ref[...], bhh_ref[...])
    h_out_ref[...] = hn
    m_out_ref[...] = jnp.dot(hn, wn_ref[...],
                             preferred_element_type=jnp.float32)


def _gru_mm(aggp, h, w_ih, w_hh, b_ih, b_hh, w_next):
    return pl.pallas_call(
        _gru_mm_body,
        grid=(N // _RB,),
        in_specs=[
            pl.BlockSpec((2, _RB, D), lambda r: (0, r, 0)),
            pl.BlockSpec((_RB, D), lambda r: (r, 0)),
            pl.BlockSpec((3 * D, D), lambda r: (0, 0)),
            pl.BlockSpec((3 * D, D), lambda r: (0, 0)),
            pl.BlockSpec((1, 3 * D), lambda r: (0, 0)),
            pl.BlockSpec((1, 3 * D), lambda r: (0, 0)),
            pl.BlockSpec((D, D), lambda r: (0, 0)),
        ],
        out_specs=[
            pl.BlockSpec((_RB, D), lambda r: (r, 0)),
            pl.BlockSpec((_RB, D), lambda r: (r, 0)),
        ],
        out_shape=[
            jax.ShapeDtypeStruct((N, D), jnp.float32),
            jax.ShapeDtypeStruct((N, D), jnp.float32),
        ],
    )(aggp, h, w_ih, w_hh, b_ih, b_hh, w_next)


def _gru_final_body(agg_ref, h_ref, wih_ref, whh_ref, bih_ref, bhh_ref,
                    out_ref):
    hn = _gru_compute(agg_ref, h_ref[...], wih_ref[...], whh_ref[...],
                      bih_ref[...], bhh_ref[...])
    norm = jnp.maximum(jnp.sqrt(jnp.sum(hn * hn, axis=1, keepdims=True)),
                       1e-12)
    out_ref[...] = jnp.maximum(hn / norm, 0.0)


def _gru_final(aggp, h, w_ih, w_hh, b_ih, b_hh):
    return pl.pallas_call(
        _gru_final_body,
        grid=(N // _RB,),
        in_specs=[
            pl.BlockSpec((2, _RB, D), lambda r: (0, r, 0)),
            pl.BlockSpec((_RB, D), lambda r: (r, 0)),
            pl.BlockSpec((3 * D, D), lambda r: (0, 0)),
            pl.BlockSpec((3 * D, D), lambda r: (0, 0)),
            pl.BlockSpec((1, 3 * D), lambda r: (0, 0)),
            pl.BlockSpec((1, 3 * D), lambda r: (0, 0)),
        ],
        out_specs=pl.BlockSpec((_RB, D), lambda r: (r, 0)),
        out_shape=jax.ShapeDtypeStruct((N, D), jnp.float32),
    )(aggp, h, w_ih, w_hh, b_ih, b_hh)


_GPB = 8


def _pool_body(x_ref, b_ref, lw_ref, lb_ref, out_ref):
    p = pl.program_id(0)
    x = x_ref[...]
    b = b_ref[...]
    rows = []
    for j in range(_GPB):
        mask = b == (p * _GPB + j)
        mx = jnp.max(jnp.where(mask, x, -jnp.inf), axis=0, keepdims=True)
        sm = jnp.sum(jnp.where(mask, x, 0.0), axis=0, keepdims=True)
        cnt = jnp.sum(mask.astype(jnp.float32))
        rows.append(jnp.concatenate([mx, sm / jnp.maximum(cnt, 1.0)], axis=1))
    pooled = jnp.concatenate(rows, axis=0)
    out_ref[...] = lax.dot_general(pooled, lw_ref[...],
                                   (((1,), (1,)), ((), ())),
                                   preferred_element_type=jnp.float32) \
        + lb_ref[...]


def _pool(x, batch2d, lin_w, lin_b):
    return pl.pallas_call(
        _pool_body,
        grid=(NUM_GRAPHS // _GPB,),
        in_specs=[
            pl.BlockSpec((N, D), lambda g: (0, 0)),
            pl.BlockSpec((N, 1), lambda g: (0, 0)),
            pl.BlockSpec((NUM_CLASS, 2 * D), lambda g: (0, 0)),
            pl.BlockSpec((1, NUM_CLASS), lambda g: (0, 0)),
        ],
        out_specs=pl.BlockSpec((_GPB, NUM_CLASS), lambda g: (g, 0)),
        out_shape=jax.ShapeDtypeStruct((NUM_GRAPHS, NUM_CLASS), jnp.float32),
    )(x, batch2d, lin_w, lin_b)


def kernel(x, edge_index, batch, weight, w_ih, w_hh, b_ih, b_hh, lin_w, lin_b):
    ei = edge_index.astype(jnp.int32)
    pad = EP - E
    src = jnp.concatenate([ei[0], jnp.zeros((pad,), jnp.int32)])
    dst = jnp.concatenate(
        [ei[1], N + (jnp.arange(pad, dtype=jnp.int32) % 16)])
    ei_p = jnp.stack([src.reshape(EP // KE, KE),
                      dst.reshape(EP // KE, KE)], axis=1)
    zeros = jnp.zeros((ZROWS, D), jnp.float32)
    batch2d = batch.astype(jnp.int32).reshape(N, 1)
    b_ih2 = b_ih.reshape(1, 3 * D)
    b_hh2 = b_hh.reshape(1, 3 * D)
    lin_b2 = lin_b.reshape(1, NUM_CLASS)

    h = x
    m = _mm0(x, weight[0])
    for i in range(3):
        aggp = _sc_agg(m, ei_p, zeros)
        if i < 2:
            h, m = _gru_mm(aggp, h, w_ih, w_hh, b_ih2, b_hh2, weight[i + 1])
        else:
            out = _gru_final(aggp, h, w_ih, w_hh, b_ih2, b_hh2)
    return _pool(out, batch2d, lin_w, lin_b2)


# speedup vs baseline: 3.4186x; 1.2395x over previous
import functools

import jax
import jax.numpy as jnp
from jax import lax
from jax.experimental import pallas as pl
from jax.experimental.pallas import tpu as pltpu
from jax.experimental.pallas import tpu_sc as plsc

N = 10000
E = 320000
D = 128
NUM_GRAPHS = 64
NUM_CLASS = 10

NTILES = 16
KE = 128
EP = 327680
EPC = EP // 2
EPT = EPC // NTILES
NCHUNK0 = 140
NCHUNK1 = 20
RPT = 624
NPAD = N + 16
ZROWS = RPT


def _sc_agg_body(m_hbm, ei_hbm, z_hbm, out_hbm, agg_s, idxr, rows,
                 sem_i, sem_g, sem_s):
    c = lax.axis_index("c")
    s = lax.axis_index("s")
    r0 = s * RPT

    pltpu.sync_copy(z_hbm.at[pl.ds(0, RPT)], agg_s.at[pl.ds(r0, RPT)])

    @pl.when(s == 0)
    def _():
        pltpu.sync_copy(z_hbm.at[pl.ds(0, 32)],
                        agg_s.at[pl.ds(16 * RPT, 32)])

    plsc.subcore_barrier()


    def g_start(q, b):
        pltpu.async_copy(m_hbm.at[idxr.at[q, 0]], rows.at[b], sem_g.at[b])

    def g_wait(q, b):
        pltpu.make_async_copy(m_hbm.at[idxr.at[q, 0]], rows.at[b],
                              sem_g.at[b]).wait()

    def s_start(q, b):
        pltpu.async_copy(rows.at[b], agg_s.at[idxr.at[q, 1]], sem_s.at[b],
                         add=True)

    def s_wait(q, b):
        pltpu.make_async_copy(rows.at[b], agg_s.at[idxr.at[q, 1]],
                              sem_s.at[b]).wait()

    def run_edges(nchunk, cb):
        def idx_start(i, q):
            pltpu.async_copy(ei_hbm.at[cb + i], idxr.at[q], sem_i.at[q])

        def idx_wait(q):
            pltpu.make_async_copy(ei_hbm.at[cb], idxr.at[q],
                                  sem_i.at[q]).wait()

        def chunk_steps(i, im, first, no_pref, no_g):
            q, q1, q2 = im % 4, (im + 1) % 4, (im + 2) % 4
            b, b1 = im % 3, (im + 1) % 3
            if not first:
                s_wait(q2, b1)
            if not no_pref:
                idx_start(i + 2, q2)
            if not no_g:
                idx_wait(q1)
                g_start(q1, b1)
            g_wait(q, b)
            s_start(q, b)

        idx_start(0, 0)
        idx_start(1, 1)
        idx_wait(0)
        g_start(0, 0)
        for k in range(4):
            chunk_steps(k, k, first=(k < 2), no_pref=False, no_g=False)

        def group(jj, carry):
            i0 = 4 + jj * 12
            for k in range(12):
                chunk_steps(i0 + k, 4 + k, first=False, no_pref=False,
                            no_g=False)
            return carry

        lax.fori_loop(0, (nchunk - 8) // 12, group, 0)

        for k in range(4):
            i = nchunk - 4 + k
            chunk_steps(i, i, first=False, no_pref=(i + 2 >= nchunk),
                        no_g=(i + 1 >= nchunk))
        for i in (nchunk - 2, nchunk - 1):
            s_wait(i % 4, i % 3)

    @pl.when(c == 0)
    def _():
        run_edges(NCHUNK0, s * NCHUNK0)

    @pl.when(c == 1)
    def _():
        run_edges(NCHUNK1, NTILES * NCHUNK0 + s * NCHUNK1)

    plsc.subcore_barrier()

    pltpu.sync_copy(agg_s.at[pl.ds(r0, RPT)], out_hbm.at[c, pl.ds(r0, RPT)])

    @pl.when(s == 15)
    def _():
        pltpu.sync_copy(agg_s.at[pl.ds(16 * RPT, 16)],
                        out_hbm.at[c, pl.ds(16 * RPT, 16)])


@functools.cache
def _sc_agg_kernel():
    return pl.kernel(
        _sc_agg_body,
        out_type=jax.ShapeDtypeStruct((2, N, D), jnp.float32),
        mesh=plsc.VectorSubcoreMesh(core_axis_name="c", subcore_axis_name="s"),
        scratch_types=[
            pltpu.VMEM_SHARED((NPAD, D), jnp.float32),
            pltpu.VMEM((4, 2, KE), jnp.int32),
            pltpu.VMEM((3, KE, D), jnp.float32),
            pltpu.SemaphoreType.DMA((4,)),
            pltpu.SemaphoreType.DMA((3,)),
            pltpu.SemaphoreType.DMA((3,)),
        ],
    )


def _sc_agg(m, ei_p, zeros):
    return _sc_agg_kernel()(m, ei_p, zeros)


_RB = 2000


def _mm0_body(x_ref, w_ref, m_ref):
    m_ref[...] = jnp.dot(x_ref[...], w_ref[...],
                         preferred_element_type=jnp.float32)


def _mm0(x, w):
    return pl.pallas_call(
        _mm0_body,
        grid=(N // _RB,),
        in_specs=[
            pl.BlockSpec((_RB, D), lambda r: (r, 0)),
            pl.BlockSpec((D, D), lambda r: (0, 0)),
        ],
        out_specs=pl.BlockSpec((_RB, D), lambda r: (r, 0)),
        out_shape=jax.ShapeDtypeStruct((N, D), jnp.float32),
    )(x, w)


def _gru_compute(aggp_ref, h, w_ih, w_hh, b_ih, b_hh):
    agg = aggp_ref[0, :, :].astype(jnp.float32) \
        + aggp_ref[1, :, :].astype(jnp.float32)
    gi = lax.dot_general(agg, w_ih, (((1,), (1,)), ((), ())),
                         preferred_element_type=jnp.float32) + b_ih
    gh = lax.dot_general(h, w_hh, (((1,), (1,)), ((), ())),
                         preferred_element_type=jnp.float32) + b_hh
    r = jax.nn.sigmoid(gi[:, :D] + gh[:, :D])
    z = jax.nn.sigmoid(gi[:, D:2 * D] + gh[:, D:2 * D])
    n = jnp.tanh(gi[:, 2 * D:] + r * gh[:, 2 * D:])
    return (1.0 - z) * n + z * h


def _gru_mm_body(agg_ref, h_ref, wih_ref, whh_ref, bih_ref, bhh_ref, wn_ref,
                 h_out_ref, m_out_ref):
    hn = _gru_compute(agg_ref, h_ref[...], wih_ref[...], whh_ref[...],
                      bih_ref[...], bhh_ref[...])
    h_out_ref[...] = hn
    m_out_ref[...] = jnp.dot(hn, wn_ref[...],
                             preferred_element_type=jnp.float32)


def _gru_mm(aggp, h, w_ih, w_hh, b_ih, b_hh, w_next):
    return pl.pallas_call(
        _gru_mm_body,
        grid=(N // _RB,),
        in_specs=[
            pl.BlockSpec((2, _RB, D), lambda r: (0, r, 0)),
            pl.BlockSpec((_RB, D), lambda r: (r, 0)),
            pl.BlockSpec((3 * D, D), lambda r: (0, 0)),
            pl.BlockSpec((3 * D, D), lambda r: (0, 0)),
            pl.BlockSpec((1, 3 * D), lambda r: (0, 0)),
            pl.BlockSpec((1, 3 * D), lambda r: (0, 0)),
            pl.BlockSpec((D, D), lambda r: (0, 0)),
        ],
        out_specs=[
            pl.BlockSpec((_RB, D), lambda r: (r, 0)),
            pl.BlockSpec((_RB, D), lambda r: (r, 0)),
        ],
        out_shape=[
            jax.ShapeDtypeStruct((N, D), jnp.float32),
            jax.ShapeDtypeStruct((N, D), jnp.float32),
        ],
    )(aggp, h, w_ih, w_hh, b_ih, b_hh, w_next)


def _gru_final_body(agg_ref, h_ref, wih_ref, whh_ref, bih_ref, bhh_ref,
                    out_ref):
    hn = _gru_compute(agg_ref, h_ref[...], wih_ref[...], whh_ref[...],
                      bih_ref[...], bhh_ref[...])
    norm = jnp.maximum(jnp.sqrt(jnp.sum(hn * hn, axis=1, keepdims=True)),
                       1e-12)
    out_ref[...] = jnp.maximum(hn / norm, 0.0)


def _gru_final(aggp, h, w_ih, w_hh, b_ih, b_hh):
    return pl.pallas_call(
        _gru_final_body,
        grid=(N // _RB,),
        in_specs=[
            pl.BlockSpec((2, _RB, D), lambda r: (0, r, 0)),
            pl.BlockSpec((_RB, D), lambda r: (r, 0)),
            pl.BlockSpec((3 * D, D), lambda r: (0, 0)),
            pl.BlockSpec((3 * D, D), lambda r: (0, 0)),
            pl.BlockSpec((1, 3 * D), lambda r: (0, 0)),
            pl.BlockSpec((1, 3 * D), lambda r: (0, 0)),
        ],
        out_specs=pl.BlockSpec((_RB, D), lambda r: (r, 0)),
        out_shape=jax.ShapeDtypeStruct((N, D), jnp.float32),
    )(aggp, h, w_ih, w_hh, b_ih, b_hh)


_GPB = 8


def _pool_body(x_ref, b_ref, lw_ref, lb_ref, out_ref):
    p = pl.program_id(0)
    x = x_ref[...]
    b = b_ref[...]
    rows = []
    for j in range(_GPB):
        mask = b == (p * _GPB + j)
        mx = jnp.max(jnp.where(mask, x, -jnp.inf), axis=0, keepdims=True)
        sm = jnp.sum(jnp.where(mask, x, 0.0), axis=0, keepdims=True)
        cnt = jnp.sum(mask.astype(jnp.float32))
        rows.append(jnp.concatenate([mx, sm / jnp.maximum(cnt, 1.0)], axis=1))
    pooled = jnp.concatenate(rows, axis=0)
    out_ref[...] = lax.dot_general(pooled, lw_ref[...],
                                   (((1,), (1,)), ((), ())),
                                   preferred_element_type=jnp.float32) \
        + lb_ref[...]


def _pool(x, batch2d, lin_w, lin_b):
    return pl.pallas_call(
        _pool_body,
        grid=(NUM_GRAPHS // _GPB,),
        in_specs=[
            pl.BlockSpec((N, D), lambda g: (0, 0)),
            pl.BlockSpec((N, 1), lambda g: (0, 0)),
            pl.BlockSpec((NUM_CLASS, 2 * D), lambda g: (0, 0)),
            pl.BlockSpec((1, NUM_CLASS), lambda g: (0, 0)),
        ],
        out_specs=pl.BlockSpec((_GPB, NUM_CLASS), lambda g: (g, 0)),
        out_shape=jax.ShapeDtypeStruct((NUM_GRAPHS, NUM_CLASS), jnp.float32),
    )(x, batch2d, lin_w, lin_b)


def kernel(x, edge_index, batch, weight, w_ih, w_hh, b_ih, b_hh, lin_w, lin_b):
    ei = edge_index.astype(jnp.int32)
    pad = EP - E
    src = jnp.concatenate([ei[0], jnp.zeros((pad,), jnp.int32)])
    dst = jnp.concatenate(
        [ei[1], N + (jnp.arange(pad, dtype=jnp.int32) % 16)])
    ei_p = jnp.stack([src.reshape(EP // KE, KE),
                      dst.reshape(EP // KE, KE)], axis=1)
    zeros = jnp.zeros((ZROWS, D), jnp.float32)
    batch2d = batch.astype(jnp.int32).reshape(N, 1)
    b_ih2 = b_ih.reshape(1, 3 * D)
    b_hh2 = b_hh.reshape(1, 3 * D)
    lin_b2 = lin_b.reshape(1, NUM_CLASS)

    h = x
    m = _mm0(x, weight[0])
    for i in range(3):
        aggp = _sc_agg(m, ei_p, zeros)
        if i < 2:
            h, m = _gru_mm(aggp, h, w_ih, w_hh, b_ih2, b_hh2, weight[i + 1])
        else:
            out = _gru_final(aggp, h, w_ih, w_hh, b_ih2, b_hh2)
    return _pool(out, batch2d, lin_w, lin_b2)
